# baseline (speedup 1.0000x reference)
import functools

import jax
import jax.numpy as jnp
from jax import lax
from jax.experimental import pallas as pl
from jax.experimental.pallas import tpu as pltpu
from jax.experimental.pallas import tpu_sc as plsc

N_STATIC = 4096
MAX_D = 512
NUM_BUCKETS = 2 * MAX_D - 1
GW = 2 * N_STATIC
NC, NS, L = 2, 16, 16
NW = NC * NS
SLABS = N_STATIC // 8
SLABS_PER_W = SLABS // NW


def _sc_body(t_hbm, out_hbm, t_vmem, g2_vmem, sem):
    wid = lax.axis_index("s") * NC + lax.axis_index("c")
    a = wid % 16
    b = wid // 16
    off = 127 - 8 * a
    pltpu.sync_copy(t_hbm, t_vmem)

    vhi = t_vmem[pl.ds(1024, L)]
    vlo = t_vmem[pl.ds(1040, L)]
    left_lo = jnp.where(b == 0, 2048 // L, 0)
    right_hi = jnp.where(b == 0, (8064 - 4736) // L, (6016 - 4736) // L)

    for r in range(8):
        @plsc.parallel_loop(left_lo, 3456 // L, unroll=8)
        def _left(c, _r=r):
            g2_vmem[_r, pl.ds(c * L, L)] = vhi

        @plsc.parallel_loop(0, right_hi, unroll=8)
        def _right(c, _r=r):
            g2_vmem[_r, pl.ds(4736 + c * L, L)] = vlo

        @plsc.parallel_loop(0, (4736 - 3456) // L, unroll=4)
        def _mid(c, _r=r):
            q = 3456 + c * L + lax.broadcasted_iota(jnp.int32, (L,), 0)
            d = jnp.clip((N_STATIC - 1) - (q + off) + _r,
                         -(MAX_D - 1), MAX_D - 1)
            g2_vmem[_r, pl.ds(3456 + c * L, L)] = plsc.load_gather(
                t_vmem, [d + (MAX_D - 1)])

    def _slab_copy(m):
        k = 16 * (b * 16 + m) + a
        s_m = pl.multiple_of(3968 - 128 * (b * 16 + m), 128)
        return pltpu.make_async_copy(
            g2_vmem.at[:, pl.ds(s_m, N_STATIC)],
            out_hbm.at[pl.ds(pl.multiple_of(8 * k, 8), 8), :],
            sem)

    _slab_copy(0).start()

    def slab(m, carry):
        _slab_copy(m).start()
        _slab_copy(m - 1).wait()
        return carry

    lax.fori_loop(1, SLABS_PER_W, slab, 0)
    _slab_copy(SLABS_PER_W - 1).wait()


@jax.jit
def _rel_pos_bias(t_pad):
    kern = pl.kernel(
        _sc_body,
        out_type=jax.ShapeDtypeStruct((N_STATIC, N_STATIC), jnp.float32),
        mesh=plsc.VectorSubcoreMesh(core_axis_name="c", subcore_axis_name="s"),
        scratch_types=[
            pltpu.VMEM((1056,), jnp.float32),
            pltpu.VMEM((8, GW), jnp.float32),
            pltpu.SemaphoreType.DMA,
        ],
        compiler_params=pltpu.CompilerParams(
            needs_layout_passes=False,
            skip_device_barrier=True,
            disable_bounds_checks=True,
            disable_semaphore_checks=True,
        ),
    )
    return kern(t_pad)


def kernel(N, emb_weight):
    t = emb_weight.reshape(-1)
    t_pad = jnp.concatenate([
        t, jnp.zeros((1,), jnp.float32),
        jnp.broadcast_to(t[NUM_BUCKETS - 1], (16,)),
        jnp.broadcast_to(t[0], (16,)),
    ])
    return _rel_pos_bias(t_pad)


# speedup vs baseline: 1.0608x; 1.0608x over previous
import jax
import jax.numpy as jnp
from jax import lax
from jax.experimental import pallas as pl
from jax.experimental.pallas import tpu as pltpu
from jax.experimental.pallas import tpu_sc as plsc

N_STATIC = 4096
MAX_D = 512
NUM_BUCKETS = 2 * MAX_D - 1
GW = 2 * N_STATIC
NC, NS, L = 2, 16, 16
NW = NC * NS
SLABS_PER_W = N_STATIC // 8 // NW
MID_LO, MID_HI = 3456, 4736


def _sc_body(t_hbm, out_hbm, t_vmem, g2_vmem, sem):
    wid = lax.axis_index("s") * NC + lax.axis_index("c")
    a = wid % 16
    b = wid // 16
    off = 127 - 8 * a
    pltpu.sync_copy(t_hbm, t_vmem.at[pl.ds(8, NUM_BUCKETS)])

    lanes = lax.broadcasted_iota(jnp.int32, (L,), 0)
    vhi = plsc.load_gather(
        t_vmem, [jnp.minimum(lanes + (NUM_BUCKETS - 1 + 8), NUM_BUCKETS - 1 + 8)])
    vlo = plsc.load_gather(t_vmem, [jnp.minimum(lanes + 8, 8)])

    for r in range(8):
        @plsc.parallel_loop(0, (MID_HI - MID_LO) // L, unroll=4)
        def _mid(c, _r=r):
            q = MID_LO + c * L + lax.broadcasted_iota(jnp.int32, (L,), 0)
            d = jnp.clip((N_STATIC - 1) - (q + off) + _r,
                         -(MAX_D - 1), MAX_D - 1)
            g2_vmem[_r, pl.ds(MID_LO + c * L, L)] = plsc.load_gather(
                t_vmem, [d + (MAX_D - 1 + 8)])

    base_a = jnp.where(b == 0, MID_HI, 0)
    trip_a = jnp.where(b == 0, (8064 - MID_HI) // L, MID_LO // L)
    val_a = jnp.where(b == 0, vlo, vhi)
    base_b2 = jnp.where(b == 0, 2048, MID_HI)
    trip_b2 = jnp.where(b == 0, (MID_LO - 2048) // L, (6016 - MID_HI) // L)
    val_b2 = jnp.where(b == 0, vhi, vlo)

    for r in range(8):
        @plsc.parallel_loop(0, trip_a, unroll=8)
        def _fill_a(c, _r=r):
            g2_vmem[_r, pl.ds(base_a + c * L, L)] = val_a

    def _slab_copy(mm):
        m = b * 16 + mm
        k = 16 * m + a
        s_m = pl.multiple_of(3968 - 128 * m, 128)
        return pltpu.make_async_copy(
            g2_vmem.at[:, pl.ds(s_m, N_STATIC)],
            out_hbm.at[pl.ds(pl.multiple_of(8 * k, 8), 8), :],
            sem)

    def early(j, carry):
        _slab_copy(jnp.where(b == 0, j, 11 + j)).start()
        return carry

    lax.fori_loop(0, 5, early, 0)

    for r in range(8):
        @plsc.parallel_loop(0, trip_b2, unroll=8)
        def _fill_b(c, _r=r):
            g2_vmem[_r, pl.ds(base_b2 + c * L, L)] = val_b2

    def late(j, carry):
        _slab_copy(jnp.where(b == 0, 5 + j, j)).start()
        _slab_copy(0).wait()
        return carry

    lax.fori_loop(0, 11, late, 0)

    def drain(j, carry):
        _slab_copy(0).wait()
        return carry

    lax.fori_loop(0, 5, drain, 0)


@jax.jit
def _rel_pos_bias(t):
    kern = pl.kernel(
        _sc_body,
        out_type=jax.ShapeDtypeStruct((N_STATIC, N_STATIC), jnp.float32),
        mesh=plsc.VectorSubcoreMesh(core_axis_name="c", subcore_axis_name="s"),
        scratch_types=[
            pltpu.VMEM((1040,), jnp.float32),
            pltpu.VMEM((8, GW), jnp.float32),
            pltpu.SemaphoreType.DMA,
        ],
        compiler_params=pltpu.CompilerParams(needs_layout_passes=False),
    )
    return kern(t)


def kernel(N, emb_weight):
    return _rel_pos_bias(emb_weight.reshape(-1))
